# baseline (speedup 1.0000x reference)
import jax
import jax.numpy as jnp
from jax.experimental import pallas as pl
from jax.experimental.pallas import tpu as pltpu

_B, _L, _H = 4, 2048, 768
_R = 16
_A = 256
_RD = 100
_RDP = 128


def _fused(x_ref, mean_ref, w4_ref, b4_ref,
           sem_rel_ref, sem_wr_W_ref, sem_wr_b_ref, sem_wg_W_ref,
           sem_wg_b_ref, sem_wx_W_ref, sem_wx_b_ref, sem_V_W_ref,
           sem_V_b_ref,
           tr_rel_ref, tr_wr_W_ref, tr_wr_b_ref, tr_wg_W_ref,
           tr_wg_b_ref, tr_wx_W_ref, tr_wx_b_ref, tr_V_W_ref,
           tr_V_b_ref,
           probs_ref, a_sem_ref, c_sem_ref, a_tr_ref, c_tr_ref):
    mean = mean_ref[0]
    bf = jnp.bfloat16
    x_bf = x_ref[0]

    probs_ref[0] = jax.nn.sigmoid(
        jnp.dot(x_bf, w4_ref[...].astype(bf),
                preferred_element_type=jnp.float32)
        + b4_ref[...])

    def attn(rel_ref, wr_W_ref, wr_b_ref, wg_W_ref, wg_b_ref,
             wx_W_ref, wx_b_ref, V_W_ref, V_b_ref, a_ref, c_ref):
        wxT = jax.lax.dot_general(
            wx_W_ref[...].astype(bf), x_bf, (((0,), (1,)), ((), ())),
            preferred_element_type=jnp.float32)
        wgT = jax.lax.dot_general(
            wg_W_ref[...], mean, (((0,), (1,)), ((), ())),
            preferred_element_type=jnp.float32)
        col = wgT + wg_b_ref[...] + wx_b_ref[...]
        wrT = jax.lax.dot_general(
            wr_W_ref[...], rel_ref[...], (((0,), (1,)), ((), ())),
            preferred_element_type=jnp.float32) + wr_b_ref[...]
        baseT = (wxT + col).astype(bf)
        wrT_bf = wrT.astype(bf)
        v = V_W_ref[...].astype(bf)
        lt = 256
        tiles = []
        for t in range(_L // lt):
            bt = baseT[:, t * lt:(t + 1) * lt]
            rows = []
            for r in range(_R):
                zrT = jnp.tanh(bt + wrT_bf[:, r:r + 1])
                rows.append(jnp.dot(v, zrT,
                                    preferred_element_type=jnp.float32))
            tiles.append(jnp.concatenate(rows, axis=0))
        e = jnp.concatenate(tiles, axis=1) + V_b_ref[...]
        e = e - jnp.max(e, axis=1, keepdims=True)
        ez = jnp.exp(e)
        a = ez / jnp.sum(ez, axis=1, keepdims=True)
        a_ref[0] = a
        c_ref[0] = jnp.dot(a.astype(bf), x_bf,
                           preferred_element_type=jnp.float32)

    attn(sem_rel_ref, sem_wr_W_ref, sem_wr_b_ref, sem_wg_W_ref,
         sem_wg_b_ref, sem_wx_W_ref, sem_wx_b_ref, sem_V_W_ref,
         sem_V_b_ref, a_sem_ref, c_sem_ref)
    attn(tr_rel_ref, tr_wr_W_ref, tr_wr_b_ref, tr_wg_W_ref,
         tr_wg_b_ref, tr_wx_W_ref, tr_wx_b_ref, tr_V_W_ref,
         tr_V_b_ref, a_tr_ref, c_tr_ref)


def kernel(description_embeddings, description_mean_embeddings,
           description_ids, semantic_relation_embeddings,
           transe_relation_embeddings, fh_start_W, fh_start_b, fh_end_W,
           fh_end_b, bt_start_W, bt_start_b, bt_end_W, bt_end_b, sem_wr_W,
           sem_wr_b, sem_wg_W, sem_wg_b, sem_wx_W, sem_wx_b, sem_V_W,
           sem_V_b, tr_wr_W, tr_wr_b, tr_wg_W, tr_wg_b, tr_wx_W, tr_wx_b,
           tr_V_W, tr_V_b):
    del description_ids
    x = description_embeddings.astype(jnp.bfloat16)
    mean = description_mean_embeddings.astype(jnp.float32).reshape(_B, 1, _H)

    w4 = jnp.concatenate([fh_start_W, fh_end_W, bt_start_W, bt_end_W],
                         axis=1)
    b4 = jnp.concatenate([fh_start_b, fh_end_b, bt_start_b,
                          bt_end_b]).reshape(1, 4)

    tr_rel = jnp.zeros((_R, _RDP), jnp.float32).at[:, :_RD].set(
        transe_relation_embeddings)
    tr_wr_Wp = jnp.zeros((_RDP, _A), jnp.float32).at[:_RD, :].set(tr_wr_W)

    full = lambda shape: pl.BlockSpec(shape, lambda b: (0,) * len(shape))

    out = pl.pallas_call(
        _fused,
        grid=(_B,),
        compiler_params=pltpu.CompilerParams(
            dimension_semantics=("parallel",)),
        in_specs=[
            pl.BlockSpec((1, _L, _H), lambda b: (b, 0, 0)),
            pl.BlockSpec((1, 1, _H), lambda b: (b, 0, 0)),
            full((_H, 4)),
            full((1, 4)),
            full((_R, _H)),
            full((_H, _A)), full((_A, 1)),
            full((_H, _A)), full((_A, 1)),
            full((_H, _A)), full((_A, 1)),
            full((1, _A)), full((1, 1)),
            full((_R, _RDP)),
            full((_RDP, _A)), full((_A, 1)),
            full((_H, _A)), full((_A, 1)),
            full((_H, _A)), full((_A, 1)),
            full((1, _A)), full((1, 1)),
        ],
        out_specs=[
            pl.BlockSpec((1, _L, 4), lambda b: (b, 0, 0)),
            pl.BlockSpec((1, _R, _L), lambda b: (b, 0, 0)),
            pl.BlockSpec((1, _R, _H), lambda b: (b, 0, 0)),
            pl.BlockSpec((1, _R, _L), lambda b: (b, 0, 0)),
            pl.BlockSpec((1, _R, _H), lambda b: (b, 0, 0)),
        ],
        out_shape=[
            jax.ShapeDtypeStruct((_B, _L, 4), jnp.float32),
            jax.ShapeDtypeStruct((_B, _R, _L), jnp.float32),
            jax.ShapeDtypeStruct((_B, _R, _H), jnp.float32),
            jax.ShapeDtypeStruct((_B, _R, _L), jnp.float32),
            jax.ShapeDtypeStruct((_B, _R, _H), jnp.float32),
        ],
    )(x, mean, w4, b4,
      semantic_relation_embeddings, sem_wr_W, sem_wr_b.reshape(_A, 1),
      sem_wg_W, sem_wg_b.reshape(_A, 1), sem_wx_W,
      sem_wx_b.reshape(_A, 1), sem_V_W.reshape(1, _A),
      sem_V_b.reshape(1, 1),
      tr_rel, tr_wr_Wp, tr_wr_b.reshape(_A, 1), tr_wg_W,
      tr_wg_b.reshape(_A, 1), tr_wx_W, tr_wx_b.reshape(_A, 1),
      tr_V_W.reshape(1, _A), tr_V_b.reshape(1, 1))

    probs, a_sem, c_sem, a_tr, c_tr = out
    return (probs[..., 0:1], probs[..., 1:2], probs[..., 2:3],
            probs[..., 3:4], c_sem, a_sem, c_tr, a_tr)


# speedup vs baseline: 1.1735x; 1.1735x over previous
import jax
import jax.numpy as jnp
from jax.experimental import pallas as pl
from jax.experimental.pallas import tpu as pltpu

_B, _L, _H = 4, 2048, 768
_R = 16
_A = 256
_RD = 100


def _fused(x_ref, mean_ref,
           fhs_W_ref, fhe_W_ref, bts_W_ref, bte_W_ref, b4_ref,
           sem_rel_ref, sem_wr_W_ref, sem_wr_b_ref, sem_wg_W_ref,
           sem_wg_b_ref, sem_wx_W_ref, sem_wx_b_ref, sem_V_W_ref,
           sem_V_b_ref,
           tr_rel_ref, tr_wr_W_ref, tr_wr_b_ref, tr_wg_W_ref,
           tr_wg_b_ref, tr_wx_W_ref, tr_wx_b_ref, tr_V_W_ref,
           tr_V_b_ref,
           fhs_ref, fhe_ref, bts_ref, bte_ref,
           a_sem_ref, c_sem_ref, a_tr_ref, c_tr_ref):
    mean = mean_ref[0]
    bf = jnp.bfloat16
    x_bf = x_ref[0].astype(bf)

    w4 = jnp.concatenate([fhs_W_ref[...], fhe_W_ref[...],
                          bts_W_ref[...], bte_W_ref[...]],
                         axis=1).astype(bf)
    probs = jax.nn.sigmoid(
        jnp.dot(x_bf, w4, preferred_element_type=jnp.float32)
        + b4_ref[...])
    fhs_ref[0] = probs[:, 0:1]
    fhe_ref[0] = probs[:, 1:2]
    bts_ref[0] = probs[:, 2:3]
    bte_ref[0] = probs[:, 3:4]

    def attn(rel_ref, wr_W_ref, wr_b_ref, wg_W_ref, wg_b_ref,
             wx_W_ref, wx_b_ref, V_W_ref, V_b_ref, a_ref, c_ref):
        wxT = jax.lax.dot_general(
            wx_W_ref[...].astype(bf), x_bf, (((0,), (1,)), ((), ())),
            preferred_element_type=jnp.float32)
        wgT = jax.lax.dot_general(
            wg_W_ref[...], mean, (((0,), (1,)), ((), ())),
            preferred_element_type=jnp.float32)
        col = wgT + wg_b_ref[...] + wx_b_ref[...]
        wrT = jax.lax.dot_general(
            wr_W_ref[...], rel_ref[...], (((0,), (1,)), ((), ())),
            preferred_element_type=jnp.float32) + wr_b_ref[...]
        baseT = (wxT + col).astype(bf)
        wrT_bf = wrT.astype(bf)
        v = V_W_ref[...].astype(bf)
        rows = []
        for r in range(_R):
            zrT = jnp.tanh(baseT + wrT_bf[:, r:r + 1])
            rows.append(jax.lax.dot_general(
                v, zrT, (((0,), (0,)), ((), ())),
                preferred_element_type=jnp.float32))
        e = jnp.concatenate(rows, axis=0) + V_b_ref[...]
        e = e - jnp.max(e, axis=1, keepdims=True)
        ez = jnp.exp(e)
        a = ez / jnp.sum(ez, axis=1, keepdims=True)
        a_ref[0] = a
        c_ref[0] = jnp.dot(a.astype(bf), x_bf,
                           preferred_element_type=jnp.float32)

    attn(sem_rel_ref, sem_wr_W_ref, sem_wr_b_ref, sem_wg_W_ref,
         sem_wg_b_ref, sem_wx_W_ref, sem_wx_b_ref, sem_V_W_ref,
         sem_V_b_ref, a_sem_ref, c_sem_ref)
    attn(tr_rel_ref, tr_wr_W_ref, tr_wr_b_ref, tr_wg_W_ref,
         tr_wg_b_ref, tr_wx_W_ref, tr_wx_b_ref, tr_V_W_ref,
         tr_V_b_ref, a_tr_ref, c_tr_ref)


def kernel(description_embeddings, description_mean_embeddings,
           description_ids, semantic_relation_embeddings,
           transe_relation_embeddings, fh_start_W, fh_start_b, fh_end_W,
           fh_end_b, bt_start_W, bt_start_b, bt_end_W, bt_end_b, sem_wr_W,
           sem_wr_b, sem_wg_W, sem_wg_b, sem_wx_W, sem_wx_b, sem_V_W,
           sem_V_b, tr_wr_W, tr_wr_b, tr_wg_W, tr_wg_b, tr_wx_W, tr_wx_b,
           tr_V_W, tr_V_b):
    del description_ids
    x = description_embeddings
    mean = description_mean_embeddings.reshape(_B, 1, _H)
    b4 = jnp.stack([fh_start_b, fh_end_b, bt_start_b, bt_end_b],
                   axis=1)

    full = lambda shape: pl.BlockSpec(shape, lambda b: (0,) * len(shape))

    out = pl.pallas_call(
        _fused,
        grid=(_B,),
        compiler_params=pltpu.CompilerParams(
            dimension_semantics=("parallel",)),
        in_specs=[
            pl.BlockSpec((1, _L, _H), lambda b: (b, 0, 0)),
            pl.BlockSpec((1, 1, _H), lambda b: (b, 0, 0)),
            full((_H, 1)), full((_H, 1)),
            full((_H, 1)), full((_H, 1)),
            full((1, 4)),
            full((_R, _H)),
            full((_H, _A)), full((_A, 1)),
            full((_H, _A)), full((_A, 1)),
            full((_H, _A)), full((_A, 1)),
            full((_A, 1)), full((1, 1)),
            full((_R, _RD)),
            full((_RD, _A)), full((_A, 1)),
            full((_H, _A)), full((_A, 1)),
            full((_H, _A)), full((_A, 1)),
            full((_A, 1)), full((1, 1)),
        ],
        out_specs=[
            pl.BlockSpec((1, _L, 1), lambda b: (b, 0, 0)),
            pl.BlockSpec((1, _L, 1), lambda b: (b, 0, 0)),
            pl.BlockSpec((1, _L, 1), lambda b: (b, 0, 0)),
            pl.BlockSpec((1, _L, 1), lambda b: (b, 0, 0)),
            pl.BlockSpec((1, _R, _L), lambda b: (b, 0, 0)),
            pl.BlockSpec((1, _R, _H), lambda b: (b, 0, 0)),
            pl.BlockSpec((1, _R, _L), lambda b: (b, 0, 0)),
            pl.BlockSpec((1, _R, _H), lambda b: (b, 0, 0)),
        ],
        out_shape=[
            jax.ShapeDtypeStruct((_B, _L, 1), jnp.float32),
            jax.ShapeDtypeStruct((_B, _L, 1), jnp.float32),
            jax.ShapeDtypeStruct((_B, _L, 1), jnp.float32),
            jax.ShapeDtypeStruct((_B, _L, 1), jnp.float32),
            jax.ShapeDtypeStruct((_B, _R, _L), jnp.float32),
            jax.ShapeDtypeStruct((_B, _R, _H), jnp.float32),
            jax.ShapeDtypeStruct((_B, _R, _L), jnp.float32),
            jax.ShapeDtypeStruct((_B, _R, _H), jnp.float32),
        ],
    )(x, mean,
      fh_start_W, fh_end_W, bt_start_W, bt_end_W, b4,
      semantic_relation_embeddings, sem_wr_W, sem_wr_b.reshape(_A, 1),
      sem_wg_W, sem_wg_b.reshape(_A, 1), sem_wx_W,
      sem_wx_b.reshape(_A, 1), sem_V_W, sem_V_b.reshape(1, 1),
      transe_relation_embeddings, tr_wr_W, tr_wr_b.reshape(_A, 1),
      tr_wg_W, tr_wg_b.reshape(_A, 1), tr_wx_W, tr_wx_b.reshape(_A, 1),
      tr_V_W, tr_V_b.reshape(1, 1))

    (fhs, fhe, bts, bte, a_sem, c_sem, a_tr, c_tr) = out
    return (fhs, fhe, bts, bte, c_sem, a_sem, c_tr, a_tr)
